# baseline (speedup 1.0000x reference)
import jax
import jax.numpy as jnp
from jax import lax
from jax.experimental import pallas as pl
from jax.experimental.pallas import tpu as pltpu

_TM = 512
_TK = 1024


def _agg_kernel(a_ref, e_ref, w_ref, b_ref, out_ref, ebf_ref, acc_ref):
    i = pl.program_id(0)
    k = pl.program_id(1)
    nk = pl.num_programs(1)

    @pl.when((i == 0) & (k == 0))
    def _():
        ebf_ref[...] = e_ref[...].astype(jnp.bfloat16)

    a_bf = a_ref[...].astype(jnp.bfloat16)
    partial = jnp.dot(a_bf, ebf_ref[pl.ds(k * _TK, _TK), :],
                      preferred_element_type=jnp.float32)

    @pl.when(k == 0)
    def _():
        acc_ref[...] = partial

    @pl.when(k != 0)
    def _():
        acc_ref[...] = acc_ref[...] + partial

    @pl.when(k == nk - 1)
    def _():
        h = acc_ref[...] + e_ref[pl.ds(i * _TM, _TM), :]
        h_bf = h.astype(jnp.bfloat16)
        w_bf = w_ref[...].astype(jnp.bfloat16)
        o = lax.dot_general(h_bf, w_bf, (((1,), (1,)), ((), ())),
                            preferred_element_type=jnp.float32)
        o = o + b_ref[...]
        out_ref[...] = jnp.where(o >= 0, o, 0.01 * o)


@jax.jit
def kernel(ego_embeddings, A_in, W, b):
    n, in_dim = ego_embeddings.shape
    out_dim = W.shape[0]
    b2 = b.reshape(1, out_dim)
    grid = (n // _TM, n // _TK)
    return pl.pallas_call(
        _agg_kernel,
        grid=grid,
        in_specs=[
            pl.BlockSpec((_TM, _TK), lambda i, k: (i, k)),
            pl.BlockSpec((n, in_dim), lambda i, k: (0, 0)),
            pl.BlockSpec((out_dim, in_dim), lambda i, k: (0, 0)),
            pl.BlockSpec((1, out_dim), lambda i, k: (0, 0)),
        ],
        out_specs=pl.BlockSpec((_TM, out_dim), lambda i, k: (i, 0)),
        out_shape=jax.ShapeDtypeStruct((n, out_dim), jnp.float32),
        scratch_shapes=[
            pltpu.VMEM((n, in_dim), jnp.bfloat16),
            pltpu.VMEM((_TM, out_dim), jnp.float32),
        ],
        compiler_params=pltpu.CompilerParams(
            dimension_semantics=("arbitrary", "arbitrary"),
        ),
    )(A_in, ego_embeddings, W, b2)


# speedup vs baseline: 1.6533x; 1.6533x over previous
import jax
import jax.numpy as jnp
from jax import lax
from jax.experimental import pallas as pl
from jax.experimental.pallas import tpu as pltpu

_TM = 512


def _agg_kernel(a_ref, e_ref, w_ref, b_ref, out_ref):
    i = pl.program_id(0)
    side = jnp.dot(a_ref[...], e_ref[...], preferred_element_type=jnp.float32)
    h = side + e_ref[pl.ds(i * _TM, _TM), :]
    o = lax.dot_general(h, w_ref[...], (((1,), (1,)), ((), ())),
                        preferred_element_type=jnp.float32)
    o = o + b_ref[...]
    out_ref[...] = jnp.where(o >= 0, o, 0.01 * o)


@jax.jit
def kernel(ego_embeddings, A_in, W, b):
    n, in_dim = ego_embeddings.shape
    out_dim = W.shape[0]
    b2 = b.reshape(1, out_dim)
    grid = (n // _TM,)
    return pl.pallas_call(
        _agg_kernel,
        grid=grid,
        in_specs=[
            pl.BlockSpec((_TM, n), lambda i: (i, 0)),
            pl.BlockSpec((n, in_dim), lambda i: (0, 0)),
            pl.BlockSpec((out_dim, in_dim), lambda i: (0, 0)),
            pl.BlockSpec((1, out_dim), lambda i: (0, 0)),
        ],
        out_specs=pl.BlockSpec((_TM, out_dim), lambda i: (i, 0)),
        out_shape=jax.ShapeDtypeStruct((n, out_dim), jnp.float32),
        compiler_params=pltpu.CompilerParams(
            dimension_semantics=("arbitrary",),
        ),
    )(A_in, ego_embeddings, W, b2)
